# baseline (speedup 1.0000x reference)
import functools

import jax
import jax.numpy as jnp
from jax import lax
from jax.experimental import pallas as pl
from jax.experimental.pallas import tpu as pltpu
from jax.experimental.pallas import tpu_sc as plsc

N = 10000
E = 160000
G = 64
H = 512

NTILES = 16
NCORES = 2
CHUNK = 80
EPT = E // NTILES
NCH = EPT // CHUNK
DCHUNK = 40
DCH = (EPT // 2) // DCHUNK

DTILES = 10
DROWS = N // DTILES

_MESH = plsc.VectorSubcoreMesh(core_axis_name="c", subcore_axis_name="s")


def _make_agg(nb, with_deg):
    nbc = nb // NCORES
    fout = (nb + 2) * 128 if with_deg else nb * 128
    scratch = [
        pltpu.VMEM_SHARED((N, 128), jnp.float32),
        pltpu.VMEM((CHUNK, 128), jnp.float32),
        pltpu.VMEM((CHUNK, 128), jnp.float32),
        pltpu.VMEM((CHUNK, 128), jnp.float32),
        pltpu.VMEM((CHUNK,), jnp.int32),
        pltpu.VMEM((CHUNK,), jnp.int32),
        pltpu.VMEM((CHUNK,), jnp.int32),
        pltpu.VMEM((CHUNK,), jnp.int32),
        pltpu.VMEM((CHUNK,), jnp.int32),
        pltpu.VMEM((CHUNK,), jnp.int32),
        pltpu.SemaphoreType.DMA,
        pltpu.SemaphoreType.DMA,
        pltpu.SemaphoreType.DMA,
        pltpu.SemaphoreType.DMA,
        pltpu.SemaphoreType.DMA,
        pltpu.SemaphoreType.DMA,
        pltpu.SemaphoreType.DMA,
        pltpu.SemaphoreType.DMA,
        pltpu.SemaphoreType.DMA,
    ]
    if with_deg:
        scratch += [
            pltpu.VMEM((DCHUNK, 128), jnp.float32),
            pltpu.VMEM((DCHUNK,), jnp.int32),
            pltpu.VMEM((DCHUNK,), jnp.int32),
        ]

    @functools.partial(pl.kernel,
                       out_type=jax.ShapeDtypeStruct((N, fout), jnp.float32),
                       mesh=_MESH, scratch_types=scratch)
    def agg_kernel(h_hbm, idx_hbm, dst_hbm, out_hbm, acc,
                   rows0, rows1, rows2, idxv0, idxv1, idxv2,
                   dstv0, dstv1, dstv2, gsem0, gsem1, gsem2,
                   csem0, csem1, csem2, ssem0, ssem1, ssem2, *degrest):
        rows = (rows0, rows1, rows2)
        idxv = (idxv0, idxv1, idxv2)
        dstv = (dstv0, dstv1, dstv2)
        gsem = (gsem0, gsem1, gsem2)
        csem = (csem0, csem1, csem2)
        ssem = (ssem0, ssem1, ssem2)
        if with_deg:
            onesb, dstd0, dstd1 = degrest
            dstd = (dstd0, dstd1)

        c = lax.axis_index("c")
        t = lax.axis_index("s")
        tb = t * EPT
        d0 = t * DROWS

        if with_deg:
            def oloop(i, carry):
                for q in range(8):
                    onesb[i, pl.ds(q * 16, 16)] = jnp.ones((16,), jnp.float32)
                return carry
            lax.fori_loop(0, DCHUNK, oloop, 0)

        def zero_acc():
            @pl.when(t < DTILES)
            def _():
                def zl(i, carry):
                    for q in range(8):
                        rows0[i, pl.ds(q * 16, 16)] = jnp.zeros((16,),
                                                                jnp.float32)
                    return carry
                lax.fori_loop(0, CHUNK, zl, 0)
                for z in range(12):
                    pltpu.async_copy(rows0,
                                     acc.at[pl.ds(d0 + z * CHUNK, CHUNK)],
                                     csem0)
                pltpu.async_copy(rows0.at[pl.ds(0, 40)],
                                 acc.at[pl.ds(d0 + 12 * CHUNK, 40)], csem0)
                for z in range(12):
                    pltpu.make_async_copy(
                        rows0, acc.at[pl.ds(d0 + z * CHUNK, CHUNK)],
                        csem0).wait()
                pltpu.make_async_copy(
                    rows0.at[pl.ds(0, 40)],
                    acc.at[pl.ds(d0 + 12 * CHUNK, 40)], csem0).wait()
            plsc.subcore_barrier()

        def dump_acc(colblk):
            @pl.when(t < DTILES)
            def _():
                hw = []
                for z in range(13):
                    buf = rows[1 + (z % 2)]
                    sz = CHUNK if z < 12 else 40
                    sbuf = buf if sz == CHUNK else buf.at[pl.ds(0, 40)]
                    r = pl.ds(d0 + z * CHUNK, sz)
                    dst = out_hbm.at[r, pl.ds(colblk * 128, 128)]
                    if z >= 2:
                        pltpu.make_async_copy(*hw[z - 2]).wait()
                    pltpu.sync_copy(acc.at[r], sbuf)
                    pltpu.async_copy(sbuf, dst, gsem[z % 2])
                    hw.append((sbuf, dst, gsem[z % 2]))
                pltpu.make_async_copy(*hw[11]).wait()
                pltpu.make_async_copy(*hw[12]).wait()

        for k in range(nbc):
            fb = c * nbc + k
            zero_acc()

            def stage(j, b):
                e0 = tb + j * CHUNK
                pltpu.async_copy(idx_hbm.at[pl.ds(fb * E + e0, CHUNK)],
                                 idxv[b], csem[b])
                pltpu.async_copy(dst_hbm.at[pl.ds(e0, CHUNK)], dstv[b],
                                 csem[b])

            def wait_stage(j, b):
                e0 = tb + j * CHUNK
                pltpu.make_async_copy(idx_hbm.at[pl.ds(fb * E + e0, CHUNK)],
                                      idxv[b], csem[b]).wait()
                pltpu.make_async_copy(dst_hbm.at[pl.ds(e0, CHUNK)], dstv[b],
                                      csem[b]).wait()

            def fire_gather(b):
                pltpu.async_copy(h_hbm.at[idxv[b]], rows[b], gsem[b])

            def wait_gather(b):
                pltpu.make_async_copy(h_hbm.at[idxv[b]], rows[b],
                                      gsem[b]).wait()

            def fire_scatter(b):
                pltpu.async_copy(rows[b], acc.at[dstv[b]], ssem[b], add=True)

            def wait_scatter(b):
                pltpu.make_async_copy(rows[b], acc.at[dstv[b]],
                                      ssem[b]).wait()

            def body(j, s, first, stage_next, gather_next):
                wait_gather(s)
                fire_scatter(s)
                s2 = (s + 2) % 3
                if not first:
                    wait_scatter(s2)
                if stage_next:
                    stage(j + 2, s2)
                if gather_next:
                    s1 = (s + 1) % 3
                    wait_stage(j + 1, s1)
                    fire_gather(s1)

            stage(0, 0)
            wait_stage(0, 0)
            fire_gather(0)
            stage(1, 1)
            body(0, 0, True, True, True)
            body(1, 1, False, True, True)
            body(2, 2, False, True, True)

            def outer(j0, carry):
                for b in range(3):
                    body(j0 * 3 + b, b, False, True, True)
                return carry
            lax.fori_loop(1, (NCH - 2) // 3, outer, 0)
            body(NCH - 2, (NCH - 2) % 3, False, False, True)
            body(NCH - 1, (NCH - 1) % 3, False, False, False)
            wait_scatter((NCH - 1) % 3)

            plsc.subcore_barrier()
            dump_acc(fb)

        if with_deg:
            zero_acc()
            db = c * (E // 2) + t * (EPT // 2)

            def dstage(j, b):
                e0 = db + j * DCHUNK
                pltpu.sync_copy(dst_hbm.at[pl.ds(e0, DCHUNK)], dstd[b])
                pltpu.async_copy(onesb, acc.at[dstd[b]], gsem[b], add=True)

            def dconsume(b):
                pltpu.make_async_copy(onesb, acc.at[dstd[b]], gsem[b]).wait()

            dstage(0, 0)

            def douter(j0, carry):
                for b in range(2):
                    dstage(j0 * 2 + b + 1, (b + 1) % 2)
                    dconsume(b)
                return carry
            lax.fori_loop(0, (DCH - 1) // 2, douter, 0)
            dconsume((DCH - 1) % 2)

            plsc.subcore_barrier()
            dump_acc(nb + c)

    return agg_kernel


_agg2 = _make_agg(2, True)
_agg4 = _make_agg(4, False)


BN = 2000


def _tc_body(agg_ref, h_ref, deg_ref, wrel_ref, wroot_ref, b_ref, out_ref):
    deg = deg_ref[:, 0:1] + deg_ref[:, 128:129]
    inv = 1.0 / jnp.maximum(deg, 1.0)
    aggn = agg_ref[:, :] * inv
    acc = lax.dot_general(aggn, wrel_ref[:, :], (((1,), (1,)), ((), ())),
                          preferred_element_type=jnp.float32,
                          precision=lax.Precision.HIGHEST)
    acc = acc + lax.dot_general(h_ref[:, :], wroot_ref[:, :],
                                (((1,), (1,)), ((), ())),
                                preferred_element_type=jnp.float32,
                                precision=lax.Precision.HIGHEST)
    out_ref[:, :] = jnp.maximum(acc + b_ref[:, :], 0.0)


def _tc_layer(agg, F, h, dega, W_rel, W_root, b):
    return pl.pallas_call(
        _tc_body,
        grid=(N // BN,),
        in_specs=[
            pl.BlockSpec((BN, F), lambda i: (i, 0)),
            pl.BlockSpec((BN, h.shape[1]), lambda i: (i, 0)),
            pl.BlockSpec((BN, 256), lambda i: (i, 1)),
            pl.BlockSpec((H, F), lambda i: (0, 0)),
            pl.BlockSpec((H, F), lambda i: (0, 0)),
            pl.BlockSpec((1, H), lambda i: (0, 0)),
        ],
        out_specs=pl.BlockSpec((BN, H), lambda i: (i, 0)),
        out_shape=jax.ShapeDtypeStruct((N, H), jnp.float32),
    )(agg, h, dega, W_rel, W_root, b)


PCHUNK = 80
PNCH = N // PCHUNK
FH = H // NCORES
PTILES = 8
PROWS = G // PTILES

_POOL_SCRATCH = [
    pltpu.VMEM_SHARED((G, 128), jnp.float32),
    pltpu.VMEM_SHARED((G, 128), jnp.float32),
    pltpu.VMEM_SHARED((G, 128), jnp.float32),
    pltpu.VMEM((PCHUNK, 128), jnp.float32),
    pltpu.VMEM((PCHUNK, 128), jnp.float32),
    pltpu.VMEM((PCHUNK,), jnp.int32),
    pltpu.VMEM((PROWS, 128), jnp.float32),
    pltpu.VMEM((PROWS, 128), jnp.float32),
    pltpu.VMEM((PCHUNK, 128), jnp.float32),
    pltpu.VMEM((PROWS, 128), jnp.float32),
]


@functools.partial(pl.kernel,
                   out_type=jax.ShapeDtypeStruct((G, H), jnp.float32),
                   mesh=_MESH, scratch_types=_POOL_SCRATCH)
def _sc_pool(h_hbm, b_hbm, out_hbm, acc0, acc1, cnt, hv0, hv1, bidx,
             av0, av1, onesv, cv):
    c = lax.axis_index("c")
    t = lax.axis_index("s")
    col0 = c * FH

    def zloop(i, carry):
        for q in range(8):
            av0[i, pl.ds(q * 16, 16)] = jnp.zeros((16,), jnp.float32)
            av1[i, pl.ds(q * 16, 16)] = jnp.zeros((16,), jnp.float32)
            cv[i, pl.ds(q * 16, 16)] = jnp.zeros((16,), jnp.float32)
        return carry
    lax.fori_loop(0, PROWS, zloop, 0)

    def oloop(i, carry):
        for q in range(8):
            onesv[i, pl.ds(q * 16, 16)] = jnp.ones((16,), jnp.float32)
        return carry
    lax.fori_loop(0, PCHUNK, oloop, 0)

    @pl.when(t < PTILES)
    def _():
        pltpu.sync_copy(av0, acc0.at[pl.ds(t * PROWS, PROWS)])
        pltpu.sync_copy(av1, acc1.at[pl.ds(t * PROWS, PROWS)])
        pltpu.sync_copy(cv, cnt.at[pl.ds(t * PROWS, PROWS)])
    plsc.subcore_barrier()

    for k in range(8):
        j = k * NTILES + t

        @pl.when(j < PNCH)
        def _():
            n0 = j * PCHUNK
            pltpu.sync_copy(b_hbm.at[pl.ds(n0, PCHUNK)], bidx)
            pltpu.sync_copy(h_hbm.at[pl.ds(n0, PCHUNK), pl.ds(col0, 128)], hv0)
            pltpu.sync_copy(h_hbm.at[pl.ds(n0, PCHUNK), pl.ds(col0 + 128, 128)],
                            hv1)
            pltpu.sync_copy(hv0, acc0.at[bidx], add=True)
            pltpu.sync_copy(hv1, acc1.at[bidx], add=True)
            pltpu.sync_copy(onesv, cnt.at[bidx], add=True)

    plsc.subcore_barrier()

    @pl.when(t < PTILES)
    def _():
        pltpu.sync_copy(acc0.at[pl.ds(t * PROWS, PROWS)], av0)
        pltpu.sync_copy(acc1.at[pl.ds(t * PROWS, PROWS)], av1)
        pltpu.sync_copy(cnt.at[pl.ds(t * PROWS, PROWS)], cv)
        for r in range(PROWS):
            for q in range(8):
                rec = 1.0 / jnp.maximum(cv[r, pl.ds(q * 16, 16)], 1.0)
                av0[r, pl.ds(q * 16, 16)] = av0[r, pl.ds(q * 16, 16)] * rec
                av1[r, pl.ds(q * 16, 16)] = av1[r, pl.ds(q * 16, 16)] * rec
        pltpu.sync_copy(av0,
                        out_hbm.at[pl.ds(t * PROWS, PROWS), pl.ds(col0, 128)])
        pltpu.sync_copy(av1,
                        out_hbm.at[pl.ds(t * PROWS, PROWS),
                                   pl.ds(col0 + 128, 128)])


def kernel(x, edge_index, batch, W_rel0, W_root0, b0, W_rel_h, W_root_h, b_h):
    src = edge_index[0]
    dst = edge_index[1]
    idx2 = (jnp.arange(2, dtype=jnp.int32)[:, None] + src[None, :] * 2).reshape(-1)
    idx4 = (jnp.arange(4, dtype=jnp.int32)[:, None] + src[None, :] * 4).reshape(-1)

    a0 = _agg2(x.reshape(N * 2, 128), idx2, dst)
    h = _tc_layer(a0, 256, x, a0, W_rel0, W_root0, b0.reshape(1, H))
    for l in range(3):
        agg = _agg4(h.reshape(N * 4, 128), idx4, dst)
        h = _tc_layer(agg, 512, h, a0, W_rel_h[l], W_root_h[l],
                      b_h[l].reshape(1, H))
    return _sc_pool(h, batch)


# speedup vs baseline: 1.0092x; 1.0092x over previous
import functools

import jax
import jax.numpy as jnp
from jax import lax
from jax.experimental import pallas as pl
from jax.experimental.pallas import tpu as pltpu
from jax.experimental.pallas import tpu_sc as plsc

N = 10000
E = 160000
G = 64
H = 512

NTILES = 16
NCORES = 2
CHUNK = 80
EPT = E // NTILES
NCH = EPT // CHUNK
DCHUNK = 40
DCH = (EPT // 2) // DCHUNK

DTILES = 10
DROWS = N // DTILES

_MESH = plsc.VectorSubcoreMesh(core_axis_name="c", subcore_axis_name="s")


def _make_agg(nb, with_deg):
    nbc = nb // NCORES
    fout = (nb + 2) * 128 if with_deg else nb * 128
    scratch = [
        pltpu.VMEM_SHARED((N, 128), jnp.float32),
        pltpu.VMEM((CHUNK, 128), jnp.float32),
        pltpu.VMEM((CHUNK, 128), jnp.float32),
        pltpu.VMEM((CHUNK, 128), jnp.float32),
        pltpu.VMEM((256,), jnp.int32),
        pltpu.VMEM((256,), jnp.int32),
        pltpu.VMEM((256,), jnp.int32),
        pltpu.SemaphoreType.DMA,
        pltpu.SemaphoreType.DMA,
        pltpu.SemaphoreType.DMA,
        pltpu.SemaphoreType.DMA,
        pltpu.SemaphoreType.DMA,
        pltpu.SemaphoreType.DMA,
        pltpu.SemaphoreType.DMA,
        pltpu.SemaphoreType.DMA,
        pltpu.SemaphoreType.DMA,
    ]
    if with_deg:
        scratch += [
            pltpu.VMEM((DCHUNK, 128), jnp.float32),
            pltpu.VMEM((DCHUNK,), jnp.int32),
            pltpu.VMEM((DCHUNK,), jnp.int32),
        ]

    @functools.partial(pl.kernel,
                       out_type=jax.ShapeDtypeStruct((N, fout), jnp.float32),
                       mesh=_MESH, scratch_types=scratch)
    def agg_kernel(h_hbm, idx_hbm, dst_hbm, out_hbm, acc,
                   rows0, rows1, rows2, cmb0, cmb1, cmb2,
                   gsem0, gsem1, gsem2,
                   csem0, csem1, csem2, ssem0, ssem1, ssem2, *degrest):
        rows = (rows0, rows1, rows2)
        cmb = (cmb0, cmb1, cmb2)
        idxv = tuple(b.at[pl.ds(0, CHUNK)] for b in cmb)
        dstv = tuple(b.at[pl.ds(128, CHUNK)] for b in cmb)
        gsem = (gsem0, gsem1, gsem2)
        csem = (csem0, csem1, csem2)
        ssem = (ssem0, ssem1, ssem2)
        if with_deg:
            onesb, dstd0, dstd1 = degrest
            dstd = (dstd0, dstd1)

        c = lax.axis_index("c")
        t = lax.axis_index("s")
        tb = t * EPT
        d0 = t * DROWS

        if with_deg:
            def oloop(i, carry):
                for q in range(8):
                    onesb[i, pl.ds(q * 16, 16)] = jnp.ones((16,), jnp.float32)
                return carry
            lax.fori_loop(0, DCHUNK, oloop, 0)

        def zero_acc():
            @pl.when(t < DTILES)
            def _():
                def zl(i, carry):
                    for q in range(8):
                        rows0[i, pl.ds(q * 16, 16)] = jnp.zeros((16,),
                                                                jnp.float32)
                    return carry
                lax.fori_loop(0, CHUNK, zl, 0)
                for z in range(12):
                    pltpu.async_copy(rows0,
                                     acc.at[pl.ds(d0 + z * CHUNK, CHUNK)],
                                     csem0)
                pltpu.async_copy(rows0.at[pl.ds(0, 40)],
                                 acc.at[pl.ds(d0 + 12 * CHUNK, 40)], csem0)
                for z in range(12):
                    pltpu.make_async_copy(
                        rows0, acc.at[pl.ds(d0 + z * CHUNK, CHUNK)],
                        csem0).wait()
                pltpu.make_async_copy(
                    rows0.at[pl.ds(0, 40)],
                    acc.at[pl.ds(d0 + 12 * CHUNK, 40)], csem0).wait()
            plsc.subcore_barrier()

        def dump_acc(colblk):
            @pl.when(t < DTILES)
            def _():
                hw = []
                for z in range(13):
                    buf = rows[1 + (z % 2)]
                    sz = CHUNK if z < 12 else 40
                    sbuf = buf if sz == CHUNK else buf.at[pl.ds(0, 40)]
                    r = pl.ds(d0 + z * CHUNK, sz)
                    dst = out_hbm.at[r, pl.ds(colblk * 128, 128)]
                    if z >= 2:
                        pltpu.make_async_copy(*hw[z - 2]).wait()
                    pltpu.sync_copy(acc.at[r], sbuf)
                    pltpu.async_copy(sbuf, dst, gsem[z % 2])
                    hw.append((sbuf, dst, gsem[z % 2]))
                pltpu.make_async_copy(*hw[11]).wait()
                pltpu.make_async_copy(*hw[12]).wait()

        for k in range(nbc):
            fb = c * nbc + k
            zero_acc()

            def stage(j, b):
                cid = (fb * (E // CHUNK) + t * NCH + j) * 256
                pltpu.async_copy(idx_hbm.at[pl.ds(cid, 256)], cmb[b], csem[b])

            def wait_stage(j, b):
                cid = (fb * (E // CHUNK) + t * NCH + j) * 256
                pltpu.make_async_copy(idx_hbm.at[pl.ds(cid, 256)], cmb[b],
                                      csem[b]).wait()

            def fire_gather(b):
                pltpu.async_copy(h_hbm.at[idxv[b]], rows[b], gsem[b])

            def wait_gather(b):
                pltpu.make_async_copy(h_hbm.at[idxv[b]], rows[b],
                                      gsem[b]).wait()

            def fire_scatter(b):
                pltpu.async_copy(rows[b], acc.at[dstv[b]], ssem[b], add=True)

            def wait_scatter(b):
                pltpu.make_async_copy(rows[b], acc.at[dstv[b]],
                                      ssem[b]).wait()

            def body(j, s, first, stage_next, gather_next):
                wait_gather(s)
                fire_scatter(s)
                s2 = (s + 2) % 3
                if not first:
                    wait_scatter(s2)
                if stage_next:
                    stage(j + 2, s2)
                if gather_next:
                    s1 = (s + 1) % 3
                    wait_stage(j + 1, s1)
                    fire_gather(s1)

            stage(0, 0)
            wait_stage(0, 0)
            fire_gather(0)
            stage(1, 1)
            body(0, 0, True, True, True)
            body(1, 1, False, True, True)
            body(2, 2, False, True, True)

            def outer(j0, carry):
                for b in range(3):
                    body(j0 * 3 + b, b, False, True, True)
                return carry
            lax.fori_loop(1, (NCH - 2) // 3, outer, 0)
            body(NCH - 2, (NCH - 2) % 3, False, False, True)
            body(NCH - 1, (NCH - 1) % 3, False, False, False)
            wait_scatter((NCH - 1) % 3)

            plsc.subcore_barrier()
            dump_acc(fb)

        if with_deg:
            zero_acc()
            db = c * (E // 2) + t * (EPT // 2)

            def dstage(j, b):
                e0 = db + j * DCHUNK
                pltpu.sync_copy(dst_hbm.at[pl.ds(e0, DCHUNK)], dstd[b])
                pltpu.async_copy(onesb, acc.at[dstd[b]], gsem[b], add=True)

            def dconsume(b):
                pltpu.make_async_copy(onesb, acc.at[dstd[b]], gsem[b]).wait()

            dstage(0, 0)

            def douter(j0, carry):
                for b in range(2):
                    dstage(j0 * 2 + b + 1, (b + 1) % 2)
                    dconsume(b)
                return carry
            lax.fori_loop(0, (DCH - 1) // 2, douter, 0)
            dconsume((DCH - 1) % 2)

            plsc.subcore_barrier()
            dump_acc(nb + c)

    return agg_kernel


_agg2 = _make_agg(2, True)
_agg4 = _make_agg(4, False)


BN = 1000


def _tc_body(agg_ref, h_ref, deg_ref, wrel_ref, wroot_ref, b_ref, out_ref):
    deg = deg_ref[:, 0:1] + deg_ref[:, 128:129]
    inv = 1.0 / jnp.maximum(deg, 1.0)
    aggn = agg_ref[:, :] * inv
    acc = lax.dot_general(aggn, wrel_ref[:, :], (((1,), (1,)), ((), ())),
                          preferred_element_type=jnp.float32,
                          precision=lax.Precision.HIGHEST)
    acc = acc + lax.dot_general(h_ref[:, :], wroot_ref[:, :],
                                (((1,), (1,)), ((), ())),
                                preferred_element_type=jnp.float32,
                                precision=lax.Precision.HIGHEST)
    out_ref[:, :] = jnp.maximum(acc + b_ref[:, :], 0.0)


def _tc_layer(agg, F, h, dega, W_rel, W_root, b):
    return pl.pallas_call(
        _tc_body,
        grid=(N // BN,),
        in_specs=[
            pl.BlockSpec((BN, F), lambda i: (i, 0)),
            pl.BlockSpec((BN, h.shape[1]), lambda i: (i, 0)),
            pl.BlockSpec((BN, 256), lambda i: (i, 1)),
            pl.BlockSpec((H, F), lambda i: (0, 0)),
            pl.BlockSpec((H, F), lambda i: (0, 0)),
            pl.BlockSpec((1, H), lambda i: (0, 0)),
        ],
        out_specs=pl.BlockSpec((BN, H), lambda i: (i, 0)),
        out_shape=jax.ShapeDtypeStruct((N, H), jnp.float32),
    )(agg, h, dega, W_rel, W_root, b)


PCHUNK = 80
PNCH = N // PCHUNK
FH = H // NCORES
PTILES = 8
PROWS = G // PTILES

_POOL_SCRATCH = [
    pltpu.VMEM_SHARED((G, 128), jnp.float32),
    pltpu.VMEM_SHARED((G, 128), jnp.float32),
    pltpu.VMEM_SHARED((G, 128), jnp.float32),
    pltpu.VMEM((PCHUNK, 128), jnp.float32),
    pltpu.VMEM((PCHUNK, 128), jnp.float32),
    pltpu.VMEM((PCHUNK,), jnp.int32),
    pltpu.VMEM((PROWS, 128), jnp.float32),
    pltpu.VMEM((PROWS, 128), jnp.float32),
    pltpu.VMEM((PCHUNK, 128), jnp.float32),
    pltpu.VMEM((PROWS, 128), jnp.float32),
]


@functools.partial(pl.kernel,
                   out_type=jax.ShapeDtypeStruct((G, H), jnp.float32),
                   mesh=_MESH, scratch_types=_POOL_SCRATCH)
def _sc_pool(h_hbm, b_hbm, out_hbm, acc0, acc1, cnt, hv0, hv1, bidx,
             av0, av1, onesv, cv):
    c = lax.axis_index("c")
    t = lax.axis_index("s")
    col0 = c * FH

    def zloop(i, carry):
        for q in range(8):
            av0[i, pl.ds(q * 16, 16)] = jnp.zeros((16,), jnp.float32)
            av1[i, pl.ds(q * 16, 16)] = jnp.zeros((16,), jnp.float32)
            cv[i, pl.ds(q * 16, 16)] = jnp.zeros((16,), jnp.float32)
        return carry
    lax.fori_loop(0, PROWS, zloop, 0)

    def oloop(i, carry):
        for q in range(8):
            onesv[i, pl.ds(q * 16, 16)] = jnp.ones((16,), jnp.float32)
        return carry
    lax.fori_loop(0, PCHUNK, oloop, 0)

    @pl.when(t < PTILES)
    def _():
        pltpu.sync_copy(av0, acc0.at[pl.ds(t * PROWS, PROWS)])
        pltpu.sync_copy(av1, acc1.at[pl.ds(t * PROWS, PROWS)])
        pltpu.sync_copy(cv, cnt.at[pl.ds(t * PROWS, PROWS)])
    plsc.subcore_barrier()

    for k in range(8):
        j = k * NTILES + t

        @pl.when(j < PNCH)
        def _():
            n0 = j * PCHUNK
            pltpu.sync_copy(b_hbm.at[pl.ds(n0, PCHUNK)], bidx)
            pltpu.sync_copy(h_hbm.at[pl.ds(n0, PCHUNK), pl.ds(col0, 128)], hv0)
            pltpu.sync_copy(h_hbm.at[pl.ds(n0, PCHUNK), pl.ds(col0 + 128, 128)],
                            hv1)
            pltpu.sync_copy(hv0, acc0.at[bidx], add=True)
            pltpu.sync_copy(hv1, acc1.at[bidx], add=True)
            pltpu.sync_copy(onesv, cnt.at[bidx], add=True)

    plsc.subcore_barrier()

    @pl.when(t < PTILES)
    def _():
        pltpu.sync_copy(acc0.at[pl.ds(t * PROWS, PROWS)], av0)
        pltpu.sync_copy(acc1.at[pl.ds(t * PROWS, PROWS)], av1)
        pltpu.sync_copy(cnt.at[pl.ds(t * PROWS, PROWS)], cv)
        for r in range(PROWS):
            for q in range(8):
                rec = 1.0 / jnp.maximum(cv[r, pl.ds(q * 16, 16)], 1.0)
                av0[r, pl.ds(q * 16, 16)] = av0[r, pl.ds(q * 16, 16)] * rec
                av1[r, pl.ds(q * 16, 16)] = av1[r, pl.ds(q * 16, 16)] * rec
        pltpu.sync_copy(av0,
                        out_hbm.at[pl.ds(t * PROWS, PROWS), pl.ds(col0, 128)])
        pltpu.sync_copy(av1,
                        out_hbm.at[pl.ds(t * PROWS, PROWS),
                                   pl.ds(col0 + 128, 128)])


def kernel(x, edge_index, batch, W_rel0, W_root0, b0, W_rel_h, W_root_h, b_h):
    src = edge_index[0]
    dst = edge_index[1]
    nch_all = E // 80
    dstp = jnp.pad(dst.reshape(1, nch_all, 80), ((0, 0), (0, 0), (0, 48)))

    def comb(nb):
        idx = (jnp.arange(nb, dtype=jnp.int32)[:, None] + src[None, :] * nb)
        idxp = jnp.pad(idx.reshape(nb, nch_all, 80), ((0, 0), (0, 0), (0, 48)))
        return jnp.stack(
            [idxp, jnp.broadcast_to(dstp, idxp.shape)], axis=2).reshape(-1)

    idx2 = comb(2)
    idx4 = comb(4)

    a0 = _agg2(x.reshape(N * 2, 128), idx2, dst)
    h = _tc_layer(a0, 256, x, a0, W_rel0, W_root0, b0.reshape(1, H))
    for l in range(3):
        agg = _agg4(h.reshape(N * 4, 128), idx4, dst)
        h = _tc_layer(agg, 512, h, a0, W_rel_h[l], W_root_h[l],
                      b_h[l].reshape(1, H))
    return _sc_pool(h, batch)


# speedup vs baseline: 1.0726x; 1.0629x over previous
import functools

import jax
import jax.numpy as jnp
from jax import lax
from jax.experimental import pallas as pl
from jax.experimental.pallas import tpu as pltpu
from jax.experimental.pallas import tpu_sc as plsc

N = 10000
E = 160000
G = 64
H = 512

NTILES = 16
NCORES = 2
CHUNK = 80
EPT = E // NTILES
NCH = EPT // CHUNK
DCHUNK = 40
DCH = (EPT // 2) // DCHUNK

DTILES = 10
DROWS = N // DTILES

_MESH = plsc.VectorSubcoreMesh(core_axis_name="c", subcore_axis_name="s")


def _make_agg(nb, with_deg):
    nbc = nb // NCORES
    fout = (nb + 2) * 128 if with_deg else nb * 128
    scratch = [
        pltpu.VMEM_SHARED((N, 128), jnp.float32),
        pltpu.VMEM((CHUNK, 128), jnp.float32),
        pltpu.VMEM((CHUNK, 128), jnp.float32),
        pltpu.VMEM((CHUNK, 128), jnp.float32),
        pltpu.VMEM((256,), jnp.int32),
        pltpu.VMEM((256,), jnp.int32),
        pltpu.VMEM((256,), jnp.int32),
        pltpu.SemaphoreType.DMA,
        pltpu.SemaphoreType.DMA,
        pltpu.SemaphoreType.DMA,
        pltpu.SemaphoreType.DMA,
        pltpu.SemaphoreType.DMA,
        pltpu.SemaphoreType.DMA,
        pltpu.SemaphoreType.DMA,
        pltpu.SemaphoreType.DMA,
        pltpu.SemaphoreType.DMA,
    ]
    if with_deg:
        scratch += [
            pltpu.VMEM((DCHUNK, 128), jnp.float32),
            pltpu.VMEM((DCHUNK,), jnp.int32),
            pltpu.VMEM((DCHUNK,), jnp.int32),
        ]

    @functools.partial(pl.kernel,
                       out_type=jax.ShapeDtypeStruct((N, fout), jnp.float32),
                       mesh=_MESH, scratch_types=scratch)
    def agg_kernel(h_hbm, idx_hbm, dst_hbm, out_hbm, acc,
                   rows0, rows1, rows2, cmb0, cmb1, cmb2,
                   gsem0, gsem1, gsem2,
                   csem0, csem1, csem2, ssem0, ssem1, ssem2, *degrest):
        rows = (rows0, rows1, rows2)
        cmb = (cmb0, cmb1, cmb2)
        idxv = tuple(b.at[pl.ds(0, CHUNK)] for b in cmb)
        dstv = tuple(b.at[pl.ds(128, CHUNK)] for b in cmb)
        gsem = (gsem0, gsem1, gsem2)
        csem = (csem0, csem1, csem2)
        ssem = (ssem0, ssem1, ssem2)
        if with_deg:
            onesb, dstd0, dstd1 = degrest
            dstd = (dstd0, dstd1)

        c = lax.axis_index("c")
        t = lax.axis_index("s")
        tb = t * EPT
        d0 = t * DROWS

        if with_deg:
            def oloop(i, carry):
                for q in range(8):
                    onesb[i, pl.ds(q * 16, 16)] = jnp.ones((16,), jnp.float32)
                return carry
            lax.fori_loop(0, DCHUNK, oloop, 0)

        def zero_acc():
            @pl.when(t < DTILES)
            def _():
                def zl(i, carry):
                    for q in range(8):
                        rows0[i, pl.ds(q * 16, 16)] = jnp.zeros((16,),
                                                                jnp.float32)
                    return carry
                lax.fori_loop(0, CHUNK, zl, 0)
                for z in range(12):
                    pltpu.async_copy(rows0,
                                     acc.at[pl.ds(d0 + z * CHUNK, CHUNK)],
                                     csem0)
                pltpu.async_copy(rows0.at[pl.ds(0, 40)],
                                 acc.at[pl.ds(d0 + 12 * CHUNK, 40)], csem0)
                for z in range(12):
                    pltpu.make_async_copy(
                        rows0, acc.at[pl.ds(d0 + z * CHUNK, CHUNK)],
                        csem0).wait()
                pltpu.make_async_copy(
                    rows0.at[pl.ds(0, 40)],
                    acc.at[pl.ds(d0 + 12 * CHUNK, 40)], csem0).wait()
            plsc.subcore_barrier()

        def dump_acc(colblk):
            @pl.when(t < DTILES)
            def _():
                hw = []
                for z in range(13):
                    buf = rows[1 + (z % 2)]
                    sz = CHUNK if z < 12 else 40
                    sbuf = buf if sz == CHUNK else buf.at[pl.ds(0, 40)]
                    r = pl.ds(d0 + z * CHUNK, sz)
                    dst = out_hbm.at[r, pl.ds(colblk * 128, 128)]
                    if z >= 2:
                        pltpu.make_async_copy(*hw[z - 2]).wait()
                    pltpu.sync_copy(acc.at[r], sbuf)
                    pltpu.async_copy(sbuf, dst, gsem[z % 2])
                    hw.append((sbuf, dst, gsem[z % 2]))
                pltpu.make_async_copy(*hw[11]).wait()
                pltpu.make_async_copy(*hw[12]).wait()

        for k in range(nbc):
            fb = c * nbc + k
            zero_acc()

            def stage(j, b):
                cid = (fb * (E // CHUNK) + t * NCH + j) * 256
                pltpu.async_copy(idx_hbm.at[pl.ds(cid, 256)], cmb[b], csem[b])

            def wait_stage(j, b):
                cid = (fb * (E // CHUNK) + t * NCH + j) * 256
                pltpu.make_async_copy(idx_hbm.at[pl.ds(cid, 256)], cmb[b],
                                      csem[b]).wait()

            def fire_gather(b):
                pltpu.async_copy(h_hbm.at[idxv[b]], rows[b], gsem[b])

            def wait_gather(b):
                pltpu.make_async_copy(h_hbm.at[idxv[b]], rows[b],
                                      gsem[b]).wait()

            def fire_scatter(b):
                pltpu.async_copy(rows[b], acc.at[dstv[b]], ssem[b], add=True)

            def wait_scatter(b):
                pltpu.make_async_copy(rows[b], acc.at[dstv[b]],
                                      ssem[b]).wait()

            def body(j, s, first, stage_next, gather_next):
                wait_gather(s)
                fire_scatter(s)
                s2 = (s + 2) % 3
                if not first:
                    wait_scatter(s2)
                if stage_next:
                    stage(j + 2, s2)
                if gather_next:
                    s1 = (s + 1) % 3
                    wait_stage(j + 1, s1)
                    fire_gather(s1)

            stage(0, 0)
            wait_stage(0, 0)
            fire_gather(0)
            stage(1, 1)
            body(0, 0, True, True, True)
            body(1, 1, False, True, True)
            body(2, 2, False, True, True)

            def outer(j0, carry):
                for b in range(3):
                    body(j0 * 3 + b, b, False, True, True)
                return carry
            lax.fori_loop(1, (NCH - 2) // 3, outer, 0)
            body(NCH - 2, (NCH - 2) % 3, False, False, True)
            body(NCH - 1, (NCH - 1) % 3, False, False, False)
            wait_scatter((NCH - 1) % 3)

            plsc.subcore_barrier()
            dump_acc(fb)

        if with_deg:
            zero_acc()
            db = c * (E // 2) + t * (EPT // 2)

            def dstage(j, b):
                e0 = db + j * DCHUNK
                pltpu.sync_copy(dst_hbm.at[pl.ds(e0, DCHUNK)], dstd[b])
                pltpu.async_copy(onesb, acc.at[dstd[b]], gsem[b], add=True)

            def dconsume(b):
                pltpu.make_async_copy(onesb, acc.at[dstd[b]], gsem[b]).wait()

            dstage(0, 0)

            def douter(j0, carry):
                for b in range(2):
                    dstage(j0 * 2 + b + 1, (b + 1) % 2)
                    dconsume(b)
                return carry
            lax.fori_loop(0, (DCH - 1) // 2, douter, 0)
            dconsume((DCH - 1) % 2)

            plsc.subcore_barrier()
            dump_acc(nb + c)

    return agg_kernel


_agg2 = _make_agg(2, True)
_agg4 = _make_agg(4, False)


BN = 1000


def _tc_root_body(h_ref, wroot_ref, out_ref):
    out_ref[:, :] = lax.dot_general(h_ref[:, :], wroot_ref[:, :],
                                    (((1,), (1,)), ((), ())),
                                    preferred_element_type=jnp.float32,
                                    precision=lax.Precision.HIGHEST)


def _tc_fin_body(agg_ref, root_ref, deg_ref, wrel_ref, b_ref, out_ref):
    deg = deg_ref[:, 0:1] + deg_ref[:, 128:129]
    inv = 1.0 / jnp.maximum(deg, 1.0)
    aggn = agg_ref[:, :] * inv
    acc = lax.dot_general(aggn, wrel_ref[:, :], (((1,), (1,)), ((), ())),
                          preferred_element_type=jnp.float32,
                          precision=lax.Precision.HIGHEST)
    out_ref[:, :] = jnp.maximum(acc + root_ref[:, :] + b_ref[:, :], 0.0)


def _tc_root(h, W_root):
    F = h.shape[1]
    return pl.pallas_call(
        _tc_root_body,
        grid=(N // BN,),
        in_specs=[
            pl.BlockSpec((BN, F), lambda i: (i, 0)),
            pl.BlockSpec((H, F), lambda i: (0, 0)),
        ],
        out_specs=pl.BlockSpec((BN, H), lambda i: (i, 0)),
        out_shape=jax.ShapeDtypeStruct((N, H), jnp.float32),
    )(h, W_root)


def _tc_fin(agg, F, root, dega, W_rel, b):
    return pl.pallas_call(
        _tc_fin_body,
        grid=(N // BN,),
        in_specs=[
            pl.BlockSpec((BN, F), lambda i: (i, 0)),
            pl.BlockSpec((BN, H), lambda i: (i, 0)),
            pl.BlockSpec((BN, 256), lambda i: (i, 1)),
            pl.BlockSpec((H, F), lambda i: (0, 0)),
            pl.BlockSpec((1, H), lambda i: (0, 0)),
        ],
        out_specs=pl.BlockSpec((BN, H), lambda i: (i, 0)),
        out_shape=jax.ShapeDtypeStruct((N, H), jnp.float32),
    )(agg, root, dega, W_rel, b)


PCHUNK = 80
PNCH = N // PCHUNK
FH = H // NCORES
PTILES = 8
PROWS = G // PTILES

_POOL_SCRATCH = [
    pltpu.VMEM_SHARED((G, 128), jnp.float32),
    pltpu.VMEM_SHARED((G, 128), jnp.float32),
    pltpu.VMEM_SHARED((G, 128), jnp.float32),
    pltpu.VMEM((PCHUNK, 128), jnp.float32),
    pltpu.VMEM((PCHUNK, 128), jnp.float32),
    pltpu.VMEM((PCHUNK,), jnp.int32),
    pltpu.VMEM((PROWS, 128), jnp.float32),
    pltpu.VMEM((PROWS, 128), jnp.float32),
    pltpu.VMEM((PCHUNK, 128), jnp.float32),
    pltpu.VMEM((PROWS, 128), jnp.float32),
]


@functools.partial(pl.kernel,
                   out_type=jax.ShapeDtypeStruct((G, H), jnp.float32),
                   mesh=_MESH, scratch_types=_POOL_SCRATCH)
def _sc_pool(h_hbm, b_hbm, out_hbm, acc0, acc1, cnt, hv0, hv1, bidx,
             av0, av1, onesv, cv):
    c = lax.axis_index("c")
    t = lax.axis_index("s")
    col0 = c * FH

    def zloop(i, carry):
        for q in range(8):
            av0[i, pl.ds(q * 16, 16)] = jnp.zeros((16,), jnp.float32)
            av1[i, pl.ds(q * 16, 16)] = jnp.zeros((16,), jnp.float32)
            cv[i, pl.ds(q * 16, 16)] = jnp.zeros((16,), jnp.float32)
        return carry
    lax.fori_loop(0, PROWS, zloop, 0)

    def oloop(i, carry):
        for q in range(8):
            onesv[i, pl.ds(q * 16, 16)] = jnp.ones((16,), jnp.float32)
        return carry
    lax.fori_loop(0, PCHUNK, oloop, 0)

    @pl.when(t < PTILES)
    def _():
        pltpu.sync_copy(av0, acc0.at[pl.ds(t * PROWS, PROWS)])
        pltpu.sync_copy(av1, acc1.at[pl.ds(t * PROWS, PROWS)])
        pltpu.sync_copy(cv, cnt.at[pl.ds(t * PROWS, PROWS)])
    plsc.subcore_barrier()

    for k in range(8):
        j = k * NTILES + t

        @pl.when(j < PNCH)
        def _():
            n0 = j * PCHUNK
            pltpu.sync_copy(b_hbm.at[pl.ds(n0, PCHUNK)], bidx)
            pltpu.sync_copy(h_hbm.at[pl.ds(n0, PCHUNK), pl.ds(col0, 128)], hv0)
            pltpu.sync_copy(h_hbm.at[pl.ds(n0, PCHUNK), pl.ds(col0 + 128, 128)],
                            hv1)
            pltpu.sync_copy(hv0, acc0.at[bidx], add=True)
            pltpu.sync_copy(hv1, acc1.at[bidx], add=True)
            pltpu.sync_copy(onesv, cnt.at[bidx], add=True)

    plsc.subcore_barrier()

    @pl.when(t < PTILES)
    def _():
        pltpu.sync_copy(acc0.at[pl.ds(t * PROWS, PROWS)], av0)
        pltpu.sync_copy(acc1.at[pl.ds(t * PROWS, PROWS)], av1)
        pltpu.sync_copy(cnt.at[pl.ds(t * PROWS, PROWS)], cv)
        for r in range(PROWS):
            for q in range(8):
                rec = 1.0 / jnp.maximum(cv[r, pl.ds(q * 16, 16)], 1.0)
                av0[r, pl.ds(q * 16, 16)] = av0[r, pl.ds(q * 16, 16)] * rec
                av1[r, pl.ds(q * 16, 16)] = av1[r, pl.ds(q * 16, 16)] * rec
        pltpu.sync_copy(av0,
                        out_hbm.at[pl.ds(t * PROWS, PROWS), pl.ds(col0, 128)])
        pltpu.sync_copy(av1,
                        out_hbm.at[pl.ds(t * PROWS, PROWS),
                                   pl.ds(col0 + 128, 128)])


def kernel(x, edge_index, batch, W_rel0, W_root0, b0, W_rel_h, W_root_h, b_h):
    src = edge_index[0]
    dst = edge_index[1]
    nch_all = E // 80
    dstp = jnp.pad(dst.reshape(1, nch_all, 80), ((0, 0), (0, 0), (0, 48)))

    def comb(nb):
        idx = (jnp.arange(nb, dtype=jnp.int32)[:, None] + src[None, :] * nb)
        idxp = jnp.pad(idx.reshape(nb, nch_all, 80), ((0, 0), (0, 0), (0, 48)))
        return jnp.stack(
            [idxp, jnp.broadcast_to(dstp, idxp.shape)], axis=2).reshape(-1)

    idx2 = comb(2)
    idx4 = comb(4)

    r0 = _tc_root(x, W_root0)
    a0 = _agg2(x.reshape(N * 2, 128), idx2, dst)
    h = _tc_fin(a0, 256, r0, a0, W_rel0, b0.reshape(1, H))
    for l in range(3):
        r = _tc_root(h, W_root_h[l])
        agg = _agg4(h.reshape(N * 4, 128), idx4, dst)
        h = _tc_fin(agg, 512, r, a0, W_rel_h[l], b_h[l].reshape(1, H))
    return _sc_pool(h, batch)
